# Initial kernel scaffold; baseline (speedup 1.0000x reference)
import jax
import jax.numpy as jnp
from jax.experimental import pallas as pl


def kernel(h_x, t_x, h_edge_index, t_edge_index, b_edge_index, h_batch, t_batch, W_conv, b_conv, Wl, Wr, b_bi, p_topk):
    raise NotImplementedError("write your pallas kernel here")


# speedup vs baseline: 11.0518x; 11.0518x over previous
import functools

import jax
import jax.numpy as jnp
from jax import lax
from jax.experimental import pallas as pl
from jax.experimental.pallas import tpu as pltpu
from jax.experimental.pallas import tpu_sc as plsc

N = 10000
E = 320000
D_IN = 128
HID = 256
B = 200
NPG = 50
K = NPG // 2

_NC = 2
_NS = 16
_EC = 128
_NCH = E // _EC
_RPT = 640
_ZR = 80

def _mesh():
  return plsc.VectorSubcoreMesh(
      core_axis_name="c", subcore_axis_name="s", num_cores=_NC,
      num_subcores=_NS)


def _fill2d(ref, nrows, ncols, value):
  def row(r, _):
    def col(k, _):
      ref[r, pl.ds(k * 16, 16)] = jnp.full((16,), value, jnp.float32)
      return ()
    lax.fori_loop(0, ncols // 16, col, ())
    return ()
  lax.fori_loop(0, nrows, row, ())


def _count_body(h_dst, t_dst, b_dst, b_src,
                cnt_h, cnt_t, cnt_bt, cnt_bh,
                idx_v, ones_v, zbuf, acc):
  c = lax.axis_index("c")
  s = lax.axis_index("s")
  row0 = s * _RPT
  nzc = jnp.where(s == _NS - 1, (N - (_NS - 1) * _RPT) // _ZR, _RPT // _ZR)

  nch = jnp.where(s < _NCH % _NS, _NCH // _NS + 1, _NCH // _NS)

  _fill2d(zbuf, _ZR, 16, 0.0)
  _fill2d(ones_v, _EC, 16, 1.0)

  def run(idx_hbm, out_hbm):
    def zc(j, _):
      pltpu.sync_copy(zbuf, acc.at[pl.ds(row0 + j * _ZR, _ZR)])
      return ()
    lax.fori_loop(0, nzc, zc, ())
    plsc.subcore_barrier()

    def step(k, _):
      base = (s + k * _NS) * _EC
      pltpu.sync_copy(idx_hbm.at[pl.ds(base, _EC)], idx_v)
      pltpu.sync_copy(ones_v, acc.at[idx_v], add=True)
      return ()
    lax.fori_loop(0, nch, step, ())
    plsc.subcore_barrier()

    def dc(j, _):
      r = row0 + j * _ZR
      pltpu.sync_copy(acc.at[pl.ds(r, _ZR)], out_hbm.at[pl.ds(r, _ZR)])
      return ()
    lax.fori_loop(0, nzc, dc, ())
    plsc.subcore_barrier()

  @pl.when(c == 0)
  def _():
    run(h_dst, cnt_h)
    run(b_dst, cnt_bt)

  @pl.when(c == 1)
  def _():
    run(t_dst, cnt_t)
    run(b_src, cnt_bh)


def _agg_body(hs, ts, hx, tx,
              h_src, h_dst, t_src, t_dst, b_src, b_dst,
              agg_h, agg_t, agg_bt, agg_bh,
              sidx, didx, rows, zbuf, acc, sem):
  c = lax.axis_index("c")
  s = lax.axis_index("s")
  row0 = s * _RPT
  nzc = jnp.where(s == _NS - 1, (N - (_NS - 1) * _RPT) // _ZR, _RPT // _ZR)

  nch = jnp.where(s < _NCH % _NS, _NCH // _NS + 1, _NCH // _NS)

  _fill2d(zbuf, _ZR, D_IN, 0.0)

  def run(table, src, dst, out_hbm):
    def zc(j, _):
      pltpu.sync_copy(zbuf, acc.at[pl.ds(row0 + j * _ZR, _ZR)])
      return ()
    lax.fori_loop(0, nzc, zc, ())
    plsc.subcore_barrier()

    def step(k, _):
      base = (s + k * _NS) * _EC
      pltpu.sync_copy(src.at[pl.ds(base, _EC)], sidx)
      cp = pltpu.async_copy(table.at[sidx], rows, sem)
      pltpu.sync_copy(dst.at[pl.ds(base, _EC)], didx)
      cp.wait()
      pltpu.sync_copy(rows, acc.at[didx], add=True)
      return ()
    lax.fori_loop(0, nch, step, ())
    plsc.subcore_barrier()

    def dc(j, _):
      r = row0 + j * _ZR
      pltpu.sync_copy(acc.at[pl.ds(r, _ZR)], out_hbm.at[pl.ds(r, _ZR)])
      return ()
    lax.fori_loop(0, nzc, dc, ())
    plsc.subcore_barrier()

  @pl.when(c == 0)
  def _():
    run(hs, h_src, h_dst, agg_h)
    run(hx, b_src, b_dst, agg_bt)

  @pl.when(c == 1)
  def _():
    run(ts, t_src, t_dst, agg_t)
    run(tx, b_dst, b_src, agg_bh)


_f32 = jnp.float32


def _sc_counts(h_dst, t_dst, b_dst, b_src):
  out = tuple(jax.ShapeDtypeStruct((N, 16), _f32) for _ in range(4))
  return pl.kernel(
      _count_body,
      out_type=out,
      mesh=_mesh(),
      scratch_types=[
          pltpu.VMEM((_EC,), jnp.int32),
          pltpu.VMEM((_EC, 16), _f32),
          pltpu.VMEM((_ZR, 16), _f32),
          pltpu.VMEM_SHARED((N, 16), _f32),
      ],
  )(h_dst, t_dst, b_dst, b_src)


def _sc_aggs(hs, ts, hx, tx, h_src, h_dst, t_src, t_dst, b_src, b_dst):
  out = tuple(jax.ShapeDtypeStruct((N, D_IN), _f32) for _ in range(4))
  return pl.kernel(
      _agg_body,
      out_type=out,
      mesh=_mesh(),
      scratch_types=[
          pltpu.VMEM((_EC,), jnp.int32),
          pltpu.VMEM((_EC,), jnp.int32),
          pltpu.VMEM((_EC, D_IN), _f32),
          pltpu.VMEM((_ZR, D_IN), _f32),
          pltpu.VMEM_SHARED((N, D_IN), _f32),
          pltpu.SemaphoreType.DMA,
      ],
  )(hs, ts, hx, tx, h_src, h_dst, t_src, t_dst, b_src, b_dst)


_BLK = 2000


def _elu(x):
  return jnp.where(x > 0, x, jnp.exp(jnp.minimum(x, 0.0)) - 1.0)


def _prep_body(hx_in, tx_in, ch, ct, cbt, cbh,
               hx_o, tx_o, hs_o, ts_o, dh_o, dt_o, rbt_o, rbh_o):
  hx = _elu(hx_in[...])
  tx = _elu(tx_in[...])
  dh = lax.rsqrt(ch[...] + 1.0)
  dt = lax.rsqrt(ct[...] + 1.0)
  hx_o[...] = hx
  tx_o[...] = tx
  hs_o[...] = hx * dh[:, 0:1]
  ts_o[...] = tx * dt[:, 0:1]
  dh_o[...] = dh
  dt_o[...] = dt
  rbt_o[...] = 1.0 / jnp.maximum(cbt[...], 1.0)
  rbh_o[...] = 1.0 / jnp.maximum(cbh[...], 1.0)


def _tc_prep(h_x, t_x, ch, ct, cbt, cbh):
  g = N // _BLK
  row = lambda i: (i, 0)
  bs128 = pl.BlockSpec((_BLK, D_IN), row)
  bs16 = pl.BlockSpec((_BLK, 16), row)
  return pl.pallas_call(
      _prep_body,
      grid=(g,),
      in_specs=[bs128, bs128, bs16, bs16, bs16, bs16],
      out_specs=[bs128, bs128, bs128, bs128, bs16, bs16, bs16, bs16],
      out_shape=[jax.ShapeDtypeStruct((N, D_IN), _f32)] * 4
      + [jax.ShapeDtypeStruct((N, 16), _f32)] * 4,
  )(h_x, t_x, ch, ct, cbt, cbh)


def _dense_body(x, xs, agg, aggb, dinv, rb, W, bconv, Wl, Wr, bbi, out):
  d = dinv[:, 0:1]
  pre = (agg[...] + xs[...]) * d
  rep = jnp.dot(pre, W[...], preferred_element_type=_f32) + bconv[...]
  mean = aggb[...] * rb[:, 0:1]
  bi = (jnp.dot(mean, Wl[...], preferred_element_type=_f32)
        + jnp.dot(x[...], Wr[...], preferred_element_type=_f32) + bbi[...])
  out[...] = jnp.concatenate([rep, bi], axis=1)


def _tc_dense(x, xs, agg, aggb, dinv, rb, W, bconv, Wl, Wr, bbi):
  g = N // _BLK
  row = lambda i: (i, 0)
  full = lambda i: (0, 0)
  bs128 = pl.BlockSpec((_BLK, D_IN), row)
  bs16 = pl.BlockSpec((_BLK, 16), row)
  wspec = pl.BlockSpec((D_IN, D_IN), full)
  bspec = pl.BlockSpec((1, D_IN), full)
  return pl.pallas_call(
      _dense_body,
      grid=(g,),
      in_specs=[bs128, bs128, bs128, bs128, bs16, bs16,
                wspec, bspec, wspec, wspec, bspec],
      out_specs=pl.BlockSpec((_BLK, HID), row),
      out_shape=jax.ShapeDtypeStruct((N, HID), _f32),
  )(x, xs, agg, aggb, dinv, rb, W, bconv, Wl, Wr, bbi)


_GB = 40


def _pool_body(hn, p, out):
  h = hn[...]
  score = jnp.sum(h * p[...], axis=2)
  sA = score[:, :, None]
  sB = score[:, None, :]
  ii = lax.broadcasted_iota(jnp.int32, (_GB, NPG, NPG), 1)
  jj = lax.broadcasted_iota(jnp.int32, (_GB, NPG, NPG), 2)
  ahead = (sB > sA) | ((sB == sA) & (jj < ii))
  rank = jnp.sum(ahead.astype(_f32), axis=2)
  w = jnp.where(rank < float(K), jnp.tanh(score), 0.0) * (1.0 / K)
  out[...] = jnp.sum(h * w[:, :, None], axis=1)


def _tc_pool(hn3, p3):
  g = B // _GB
  return pl.pallas_call(
      _pool_body,
      grid=(g,),
      in_specs=[pl.BlockSpec((_GB, NPG, HID), lambda i: (i, 0, 0)),
                pl.BlockSpec((1, 1, HID), lambda i: (0, 0, 0))],
      out_specs=pl.BlockSpec((_GB, HID), lambda i: (i, 0)),
      out_shape=jax.ShapeDtypeStruct((B, HID), _f32),
  )(hn3, p3)


def kernel(h_x, t_x, h_edge_index, t_edge_index, b_edge_index,
           h_batch, t_batch, W_conv, b_conv, Wl, Wr, b_bi, p_topk):
  h_src, h_dst = h_edge_index[0], h_edge_index[1]
  t_src, t_dst = t_edge_index[0], t_edge_index[1]
  b_src, b_dst = b_edge_index[0], b_edge_index[1]

  cnt_h, cnt_t, cnt_bt, cnt_bh = _sc_counts(h_dst, t_dst, b_dst, b_src)
  hx, tx, hs, ts, dh, dt, rbt, rbh = _tc_prep(
      h_x, t_x, cnt_h, cnt_t, cnt_bt, cnt_bh)
  agg_h, agg_t, agg_bt, agg_bh = _sc_aggs(
      hs, ts, hx, tx, h_src, h_dst, t_src, t_dst, b_src, b_dst)

  bc = b_conv.reshape(1, D_IN)
  bb = b_bi.reshape(1, D_IN)
  h_new = _tc_dense(hx, hs, agg_h, agg_bh, dh, rbh, W_conv, bc, Wl, Wr, bb)
  t_new = _tc_dense(tx, ts, agg_t, agg_bt, dt, rbt, W_conv, bc, Wl, Wr, bb)

  p3 = (p_topk / (jnp.linalg.norm(p_topk) + 1e-16)).reshape(1, 1, HID)
  h_emb = _tc_pool(h_new.reshape(B, NPG, HID), p3)
  t_emb = _tc_pool(t_new.reshape(B, NPG, HID), p3)
  return (h_new, t_new, h_emb, t_emb)
